# baseline (speedup 1.0000x reference)
import functools

import jax
import jax.numpy as jnp
from jax import lax
from jax.experimental import pallas as pl
from jax.experimental.pallas import tpu as pltpu
from jax.experimental.pallas import tpu_sc as plsc

N = 10000
E = 320000
DIN = 128
HID = 128
DOUT = 128
T = 8

NC = 2
NS = 16
NT = NC * NS
L = 16

CH = 128
EP = 331776
NCHUNK = EP // (NT * CH)
NCHUNK2 = EP // (NS * CH)

NPAD = 10240
HALF = NPAD // 2
TPT = NPAD // NT
SPT = NPAD // NT
ZR = 32

CAPH = 12416
CAPB = CAPH + L
NCH3 = CAPH // CH

BN = 1000
BP = 1024

_MESH = plsc.VectorSubcoreMesh(
    core_axis_name="c", subcore_axis_name="s", num_cores=NC, num_subcores=NS)
_SC_PARAMS = pltpu.CompilerParams(needs_layout_passes=False)

_f32 = jnp.float32
_i32 = jnp.int32


def _mm_body(x_ref, w_ref, o_ref):
    o_ref[...] = jnp.dot(x_ref[...], w_ref[...], preferred_element_type=_f32)


def _lin_body(p_ref, b_ref, w_ref, pb_ref, f_ref, o_ref):
    z = p_ref[...] + b_ref[...]
    z = jnp.where(f_ref[0, 0] > 0.0, jnp.maximum(z, 0.0), z)
    o_ref[...] = jnp.dot(z, w_ref[...], preferred_element_type=_f32) + pb_ref[...]


def _tc_matmul(xf, wrep):
    return pl.pallas_call(
        _mm_body,
        grid=(N // BN,),
        in_specs=[
            pl.BlockSpec((BN, DIN * T), lambda i: (i, 0)),
            pl.BlockSpec((DIN * T, HID), lambda i: (0, 0)),
        ],
        out_specs=pl.BlockSpec((BN, HID), lambda i: (i, 0)),
        out_shape=jax.ShapeDtypeStruct((NPAD, HID), _f32),
    )(xf, wrep)


def _tc_lin(parts, b, w, pb, flag):
    return pl.pallas_call(
        _lin_body,
        grid=(NPAD // BP,),
        in_specs=[
            pl.BlockSpec((BP, HID), lambda i: (i, 0)),
            pl.BlockSpec((1, HID), lambda i: (0, 0)),
            pl.BlockSpec((HID, HID), lambda i: (0, 0)),
            pl.BlockSpec((1, HID), lambda i: (0, 0)),
            pl.BlockSpec((1, 1), lambda i: (0, 0)),
        ],
        out_specs=pl.BlockSpec((BP, HID), lambda i: (i, 0)),
        out_shape=jax.ShapeDtypeStruct((NPAD, HID), _f32),
    )(parts, b, w, pb, flag)


@functools.partial(
    pl.kernel,
    out_type=jax.ShapeDtypeStruct((NT * NPAD,), _f32),
    mesh=_MESH,
    compiler_params=_SC_PARAMS,
    scratch_types=[
        pltpu.VMEM((NCHUNK, CH), _i32),
        pltpu.VMEM((NCHUNK, CH), _f32),
        pltpu.VMEM((NPAD,), _f32),
    ],
)
def _sc_deg(dst_hbm, w_hbm, degp_hbm, dbuf, wbuf, degl):
    c = lax.axis_index("c")
    s = lax.axis_index("s")
    g = c * NS + s
    pltpu.sync_copy(dst_hbm.at[g], dbuf)
    pltpu.sync_copy(w_hbm.at[g], wbuf)

    def zero_body(i, _):
        degl[pl.ds(i * L, L)] = jnp.zeros((L,), _f32)
        return 0
    lax.fori_loop(0, NPAD // L, zero_body, 0)

    def scat_body(i, _):
        for k in range(CH // L):
            dv = dbuf[i, pl.ds(k * L, L)]
            wv = wbuf[i, pl.ds(k * L, L)]
            plsc.addupdate_scatter(degl, [dv], wv)
        return 0
    lax.fori_loop(0, NCHUNK, scat_body, 0)
    pltpu.sync_copy(degl, degp_hbm.at[pl.ds(g * NPAD, NPAD)])


@functools.partial(
    pl.kernel,
    out_type=jax.ShapeDtypeStruct((NPAD,), _f32),
    mesh=_MESH,
    compiler_params=_SC_PARAMS,
    scratch_types=[
        pltpu.VMEM((TPT,), _f32),
        pltpu.VMEM((TPT,), _f32),
        pltpu.VMEM((TPT,), _f32),
    ],
)
def _sc_dis(degp_hbm, dis_hbm, a, b, o):
    c = lax.axis_index("c")
    s = lax.axis_index("s")
    g = c * NS + s
    r0 = g * TPT
    pltpu.sync_copy(degp_hbm.at[pl.ds(r0, TPT)], a)
    for j in range(1, NT):
        pltpu.sync_copy(degp_hbm.at[pl.ds(j * NPAD + r0, TPT)], b)

        def add_body(i, _):
            a[pl.ds(i * L, L)] = a[pl.ds(i * L, L)] + b[pl.ds(i * L, L)]
            return 0
        lax.fori_loop(0, TPT // L, add_body, 0)

    def body(i, _):
        sl = pl.ds(i * L, L)
        x = jnp.maximum(a[sl], 1.0)
        xi = plsc.bitcast(x, _i32)
        yi = jnp.int32(0x5F3759DF) - (xi >> 1)
        y = plsc.bitcast(yi, _f32)
        for _ in range(3):
            y = y * (1.5 - 0.5 * x * y * y)
        o[sl] = y
        return 0
    lax.fori_loop(0, TPT // L, body, 0)
    pltpu.sync_copy(o, dis_hbm.at[pl.ds(r0, TPT)])


@functools.partial(
    pl.kernel,
    out_type=[
        jax.ShapeDtypeStruct((NT * CAPH,), _i32),
        jax.ShapeDtypeStruct((NT * CAPH,), _i32),
        jax.ShapeDtypeStruct((NT * CAPH,), _f32),
        jax.ShapeDtypeStruct((NT * L,), _i32),
    ],
    mesh=_MESH,
    compiler_params=_SC_PARAMS,
    scratch_types=[
        pltpu.VMEM((NCHUNK2, CH), _i32),
        pltpu.VMEM((NCHUNK2, CH), _i32),
        pltpu.VMEM((NCHUNK2, CH), _f32),
        pltpu.VMEM((NPAD,), _f32),
        pltpu.VMEM((CAPB,), _i32),
        pltpu.VMEM((CAPB,), _i32),
        pltpu.VMEM((CAPB,), _f32),
        pltpu.VMEM((L,), _i32),
    ],
)
def _sc_bin(srcp_hbm, dstp_hbm, wp_hbm, dis_hbm,
            csrc_hbm, cdl_hbm, cco_hbm, cnt_hbm,
            sb, db, wb, disb, fsrc, fdl, fco, cntb):
    c = lax.axis_index("c")
    s = lax.axis_index("s")
    g = c * NS + s
    pltpu.sync_copy(srcp_hbm.at[s], sb)
    pltpu.sync_copy(dstp_hbm.at[s], db)
    pltpu.sync_copy(wp_hbm.at[s], wb)
    pltpu.sync_copy(dis_hbm, disb)

    def zf(i, _):
        sl = pl.ds(i * L, L)
        fsrc[sl] = jnp.zeros((L,), _i32)
        fdl[sl] = jnp.zeros((L,), _i32)
        fco[sl] = jnp.zeros((L,), _f32)
        return 0
    lax.fori_loop(0, CAPB // L, zf, 0)

    lo = c * HALF

    def bin_chunk(i, off):
        for k in range(CH // L):
            sl = pl.ds(k * L, L)
            sv = sb[i, sl]
            dv = db[i, sl]
            wv = wb[i, sl]
            dsv = plsc.load_gather(disb, [sv])
            ddv = plsc.load_gather(disb, [dv])
            cv = dsv * wv * ddv
            lv = dv - lo
            own = (lv >= 0) & (lv < HALF)
            plsc.store_compressed(fsrc.at[pl.ds(off, L)], sv, mask=own)
            plsc.store_compressed(fdl.at[pl.ds(off, L)], lv, mask=own)
            plsc.store_compressed(fco.at[pl.ds(off, L)], cv, mask=own)
            cnt16 = plsc.all_reduce_population_count(own)
            off = off + cnt16[0]
        return off
    off = lax.fori_loop(0, NCHUNK2, bin_chunk, jnp.int32(0))

    cntb[...] = jnp.full((L,), off, _i32)
    pltpu.sync_copy(cntb, cnt_hbm.at[pl.ds(g * L, L)])
    pltpu.sync_copy(fsrc.at[pl.ds(0, CAPH)], csrc_hbm.at[pl.ds(g * CAPH, CAPH)])
    pltpu.sync_copy(fdl.at[pl.ds(0, CAPH)], cdl_hbm.at[pl.ds(g * CAPH, CAPH)])
    pltpu.sync_copy(fco.at[pl.ds(0, CAPH)], cco_hbm.at[pl.ds(g * CAPH, CAPH)])


@functools.partial(
    pl.kernel,
    out_type=[
        jax.ShapeDtypeStruct((NT * CAPH,), _i32),
        jax.ShapeDtypeStruct((NT * CAPH,), _i32),
        jax.ShapeDtypeStruct((NT * CAPH,), _f32),
        jax.ShapeDtypeStruct((NT * L,), _i32),
    ],
    mesh=_MESH,
    compiler_params=_SC_PARAMS,
    scratch_types=[
        pltpu.VMEM((CAPH,), _i32),
        pltpu.VMEM((CAPH,), _i32),
        pltpu.VMEM((CAPH,), _f32),
        pltpu.VMEM((CAPB,), _i32),
        pltpu.VMEM((CAPB,), _i32),
        pltpu.VMEM((CAPB,), _f32),
        pltpu.VMEM((L,), _i32),
    ],
)
def _sc_shuf(csrc_hbm, cdl_hbm, cco_hbm, cnt_hbm,
             tsrc_hbm, tdl_hbm, tco_hbm, tcnt_hbm,
             bsrc, bdl, bco, fsrc, fdl, fco, cntb):
    c = lax.axis_index("c")
    s = lax.axis_index("s")
    g = c * NS + s

    def zf(i, _):
        sl = pl.ds(i * L, L)
        fsrc[sl] = jnp.zeros((L,), _i32)
        fdl[sl] = jnp.zeros((L,), _i32)
        fco[sl] = jnp.zeros((L,), _f32)
        return 0
    lax.fori_loop(0, CAPB // L, zf, 0)

    lo = s * SPT
    off = jnp.int32(0)
    for j in range(NS):
        gj = c * NS + j
        pltpu.sync_copy(csrc_hbm.at[pl.ds(gj * CAPH, CAPH)], bsrc)
        pltpu.sync_copy(cdl_hbm.at[pl.ds(gj * CAPH, CAPH)], bdl)
        pltpu.sync_copy(cco_hbm.at[pl.ds(gj * CAPH, CAPH)], bco)
        pltpu.sync_copy(cnt_hbm.at[pl.ds(gj * L, L)], cntb)
        cntj = cntb[...][0]
        nv = (cntj + L - 1) // L

        def pick(i, o):
            sl = pl.ds(i * L, L)
            sv = bsrc[sl]
            dv = bdl[sl]
            cv = bco[sl]
            lv = dv - lo
            own = (lv >= 0) & (lv < SPT)
            plsc.store_compressed(fsrc.at[pl.ds(o, L)], sv, mask=own)
            plsc.store_compressed(fdl.at[pl.ds(o, L)], dv, mask=own)
            plsc.store_compressed(fco.at[pl.ds(o, L)], cv, mask=own)
            cnt16 = plsc.all_reduce_population_count(own)
            return o + cnt16[0]
        off = lax.fori_loop(0, nv, pick, off)

    cntb[...] = jnp.full((L,), off, _i32)
    pltpu.sync_copy(cntb, tcnt_hbm.at[pl.ds(g * L, L)])
    pltpu.sync_copy(fsrc.at[pl.ds(0, CAPH)], tsrc_hbm.at[pl.ds(g * CAPH, CAPH)])
    pltpu.sync_copy(fdl.at[pl.ds(0, CAPH)], tdl_hbm.at[pl.ds(g * CAPH, CAPH)])
    pltpu.sync_copy(fco.at[pl.ds(0, CAPH)], tco_hbm.at[pl.ds(g * CAPH, CAPH)])


@functools.partial(
    pl.kernel,
    out_type=jax.ShapeDtypeStruct((NPAD, HID), _f32),
    mesh=_MESH,
    compiler_params=_SC_PARAMS,
    scratch_types=[
        pltpu.VMEM((CAPH,), _i32),
        pltpu.VMEM((CAPH,), _i32),
        pltpu.VMEM((CAPH,), _f32),
        pltpu.VMEM((NCH3, CH), _i32),
        pltpu.VMEM((CH, HID), _f32),
        pltpu.VMEM((CH, HID), _f32),
        pltpu.VMEM((ZR, HID), _f32),
        pltpu.VMEM((L,), _i32),
        pltpu.VMEM_SHARED((HALF, HID), _f32),
        pltpu.SemaphoreType.DMA,
        pltpu.SemaphoreType.DMA,
    ],
)
def _sc_agg(hm_hbm, tsrc_hbm, tdl_hbm, tco_hbm, tcnt_hbm, parts_hbm,
            fsrc, fdl, fco, didx2d, rows0, rows1, zbuf, cntb, acc,
            sem0, sem1):
    c = lax.axis_index("c")
    s = lax.axis_index("s")
    g = c * NS + s
    pltpu.sync_copy(tsrc_hbm.at[pl.ds(g * CAPH, CAPH)], fsrc)
    pltpu.sync_copy(tdl_hbm.at[pl.ds(g * CAPH, CAPH)], fdl)
    pltpu.sync_copy(tco_hbm.at[pl.ds(g * CAPH, CAPH)], fco)
    pltpu.sync_copy(tcnt_hbm.at[pl.ds(g * L, L)], cntb)
    cnt = cntb[...][0]
    npair = jnp.minimum((cnt + 2 * CH - 1) // (2 * CH), CAPH // (2 * CH))
    nch = npair * 2

    def cp(i, _):
        for k in range(CH // L):
            didx2d[i, pl.ds(k * L, L)] = fdl[pl.ds(i * CH + k * L, L)]
        return 0
    lax.fori_loop(0, nch, cp, 0)

    def zb(i, _):
        for j in range(HID // L):
            zbuf[i, pl.ds(j * L, L)] = jnp.zeros((L,), _f32)
        return 0
    lax.fori_loop(0, ZR, zb, 0)
    r0 = s * SPT
    for kk in range(SPT // ZR):
        pltpu.sync_copy(zbuf, acc.at[pl.ds(r0 + kk * ZR, ZR)])
    plsc.subcore_barrier()

    def chunk_work(chunk, rowsb):
        def scale(k, _):
            cv = fco[pl.ds(chunk * CH + k * L, L)]
            for r in range(L):
                bv = jnp.full((L,), cv[r], _f32)
                row = k * L + r
                for j in range(HID // L):
                    rowsb[row, pl.ds(j * L, L)] = (
                        rowsb[row, pl.ds(j * L, L)] * bv)
            return 0
        lax.fori_loop(0, CH // L, scale, 0)
        pltpu.sync_copy(rowsb, acc.at[didx2d.at[chunk]], add=True)

    def gather(chunk, rowsb, semb):
        pltpu.async_copy(hm_hbm.at[fsrc.at[pl.ds(chunk * CH, CH)]],
                         rowsb, semb)

    def gwait(rowsb, semb):
        pltpu.make_async_copy(hm_hbm.at[fsrc.at[pl.ds(0, CH)]],
                              rowsb, semb).wait()

    gather(0, rows0, sem0)

    def pair(i, _):
        c0 = 2 * i
        c2 = jnp.where(c0 + 2 < nch, c0 + 2, 0)
        gwait(rows0, sem0)
        gather(c0 + 1, rows1, sem1)
        chunk_work(c0, rows0)
        gwait(rows1, sem1)
        gather(c2, rows0, sem0)
        chunk_work(c0 + 1, rows1)
        return 0
    lax.fori_loop(0, npair, pair, 0)
    gwait(rows0, sem0)

    plsc.subcore_barrier()
    pltpu.sync_copy(acc.at[pl.ds(r0, SPT)], parts_hbm.at[pl.ds(g * SPT, SPT)])


def kernel(x, edge_index, edge_attr, W1, b1, W2, b2, Wout, bout):
    xf = x.reshape(N, DIN * T)
    w1rep = jnp.repeat(W1, T, axis=0) * (1.0 / T)

    src = edge_index[0].astype(_i32)
    dst = edge_index[1].astype(_i32)
    w = edge_attr[:, -1].astype(_f32)

    loop = jnp.arange(N, dtype=_i32)
    npadE = EP - E - N
    ipad = jnp.zeros((npadE,), dtype=_i32)
    dpad = jnp.arange(npadE, dtype=_i32) * (N // npadE)
    srcf = jnp.concatenate([src, loop, ipad])
    dstf = jnp.concatenate([dst, loop, dpad])
    wf = jnp.concatenate(
        [w, jnp.ones((N,), _f32), jnp.zeros((EP - E - N,), _f32)])

    srcp = srcf.reshape(NS, NCHUNK2, CH)
    dstp = dstf.reshape(NS, NCHUNK2, CH)
    wp = wf.reshape(NS, NCHUNK2, CH)
    dstp_deg = dstf.reshape(NT, NCHUNK, CH)
    wp_deg = wf.reshape(NT, NCHUNK, CH)

    b1r = b1.reshape(1, HID)
    b2r = b2.reshape(1, HID)
    boutr = bout.reshape(1, DOUT)

    hm1 = _tc_matmul(xf, w1rep)
    degp = _sc_deg(dstp_deg, wp_deg)
    dis = _sc_dis(degp)
    csrc, cdl, cco, cnt = _sc_bin(srcp, dstp, wp, dis)
    tsrc, tdl, tco, tcnt = _sc_shuf(csrc, cdl, cco, cnt)

    ws = jnp.stack([W2, Wout])
    bs = jnp.stack([b1r, b2r])
    pbs = jnp.stack([jnp.zeros((1, HID), _f32), boutr])
    flags = jnp.array([[[1.0]], [[0.0]]], _f32)

    def stage(hm, xs):
        wk, bk, pbk, fk = xs
        parts = _sc_agg(hm, tsrc, tdl, tco, tcnt)
        hm_next = _tc_lin(parts, bk, wk, pbk, fk)
        return hm_next, 0

    out, _ = lax.scan(stage, hm1, (ws, bs, pbs, flags))
    return out[:N]


# speedup vs baseline: 1.1986x; 1.1986x over previous
import functools

import jax
import jax.numpy as jnp
from jax import lax
from jax.experimental import pallas as pl
from jax.experimental.pallas import tpu as pltpu
from jax.experimental.pallas import tpu_sc as plsc

N = 10000
E = 320000
DIN = 128
HID = 128
DOUT = 128
T = 8

NC = 2
NS = 16
NT = NC * NS
L = 16

CH = 128
EP = 331776
NCHUNK = EP // (NT * CH)
NCHUNK2 = EP // (NS * CH)

NPAD = 10240
HALF = NPAD // 2
TPT = NPAD // NT
SPT = HALF // NS
ZR = 32

CAPH = 12416
CAPB = CAPH + L
NCH3 = CAPH // CH

BN = 1000
BP = 1024

_MESH = plsc.VectorSubcoreMesh(
    core_axis_name="c", subcore_axis_name="s", num_cores=NC, num_subcores=NS)
_SC_PARAMS = pltpu.CompilerParams(needs_layout_passes=False)

_f32 = jnp.float32
_i32 = jnp.int32


def _mm_body(x_ref, w_ref, o_ref):
    o_ref[...] = jnp.dot(x_ref[...], w_ref[...], preferred_element_type=_f32)


def _lin_body(p_ref, b_ref, w_ref, pb_ref, f_ref, o_ref):
    z = p_ref[...] + b_ref[...]
    z = jnp.where(f_ref[0, 0] > 0.0, jnp.maximum(z, 0.0), z)
    o_ref[...] = jnp.dot(z, w_ref[...], preferred_element_type=_f32) + pb_ref[...]


def _tc_matmul(xf, wrep):
    return pl.pallas_call(
        _mm_body,
        grid=(N // BN,),
        in_specs=[
            pl.BlockSpec((BN, DIN * T), lambda i: (i, 0)),
            pl.BlockSpec((DIN * T, HID), lambda i: (0, 0)),
        ],
        out_specs=pl.BlockSpec((BN, HID), lambda i: (i, 0)),
        out_shape=jax.ShapeDtypeStruct((NPAD, HID), _f32),
    )(xf, wrep)


def _tc_lin(parts, b, w, pb, flag):
    return pl.pallas_call(
        _lin_body,
        grid=(NPAD // BP,),
        in_specs=[
            pl.BlockSpec((BP, HID), lambda i: (i, 0)),
            pl.BlockSpec((1, HID), lambda i: (0, 0)),
            pl.BlockSpec((HID, HID), lambda i: (0, 0)),
            pl.BlockSpec((1, HID), lambda i: (0, 0)),
            pl.BlockSpec((1, 1), lambda i: (0, 0)),
        ],
        out_specs=pl.BlockSpec((BP, HID), lambda i: (i, 0)),
        out_shape=jax.ShapeDtypeStruct((NPAD, HID), _f32),
    )(parts, b, w, pb, flag)


@functools.partial(
    pl.kernel,
    out_type=jax.ShapeDtypeStruct((NT * NPAD,), _f32),
    mesh=_MESH,
    compiler_params=_SC_PARAMS,
    scratch_types=[
        pltpu.VMEM((NCHUNK, CH), _i32),
        pltpu.VMEM((NCHUNK, CH), _f32),
        pltpu.VMEM((NPAD,), _f32),
    ],
)
def _sc_deg(dst_hbm, w_hbm, degp_hbm, dbuf, wbuf, degl):
    c = lax.axis_index("c")
    s = lax.axis_index("s")
    g = c * NS + s
    pltpu.sync_copy(dst_hbm.at[g], dbuf)
    pltpu.sync_copy(w_hbm.at[g], wbuf)

    def zero_body(i, _):
        degl[pl.ds(i * L, L)] = jnp.zeros((L,), _f32)
        return 0
    lax.fori_loop(0, NPAD // L, zero_body, 0)

    def scat_body(i, _):
        for k in range(CH // L):
            dv = dbuf[i, pl.ds(k * L, L)]
            wv = wbuf[i, pl.ds(k * L, L)]
            plsc.addupdate_scatter(degl, [dv], wv)
        return 0
    lax.fori_loop(0, NCHUNK, scat_body, 0)
    pltpu.sync_copy(degl, degp_hbm.at[pl.ds(g * NPAD, NPAD)])


@functools.partial(
    pl.kernel,
    out_type=jax.ShapeDtypeStruct((NPAD,), _f32),
    mesh=_MESH,
    compiler_params=_SC_PARAMS,
    scratch_types=[
        pltpu.VMEM((TPT,), _f32),
        pltpu.VMEM((TPT,), _f32),
        pltpu.VMEM((TPT,), _f32),
    ],
)
def _sc_dis(degp_hbm, dis_hbm, a, b, o):
    c = lax.axis_index("c")
    s = lax.axis_index("s")
    g = c * NS + s
    r0 = g * TPT
    pltpu.sync_copy(degp_hbm.at[pl.ds(r0, TPT)], a)
    for j in range(1, NT):
        pltpu.sync_copy(degp_hbm.at[pl.ds(j * NPAD + r0, TPT)], b)

        def add_body(i, _):
            a[pl.ds(i * L, L)] = a[pl.ds(i * L, L)] + b[pl.ds(i * L, L)]
            return 0
        lax.fori_loop(0, TPT // L, add_body, 0)

    def body(i, _):
        sl = pl.ds(i * L, L)
        x = jnp.maximum(a[sl], 1.0)
        xi = plsc.bitcast(x, _i32)
        yi = jnp.int32(0x5F3759DF) - (xi >> 1)
        y = plsc.bitcast(yi, _f32)
        for _ in range(3):
            y = y * (1.5 - 0.5 * x * y * y)
        o[sl] = y
        return 0
    lax.fori_loop(0, TPT // L, body, 0)
    pltpu.sync_copy(o, dis_hbm.at[pl.ds(r0, TPT)])


@functools.partial(
    pl.kernel,
    out_type=[
        jax.ShapeDtypeStruct((NT * CAPH,), _i32),
        jax.ShapeDtypeStruct((NT * CAPH,), _i32),
        jax.ShapeDtypeStruct((NT * CAPH,), _f32),
        jax.ShapeDtypeStruct((NT * L,), _i32),
    ],
    mesh=_MESH,
    compiler_params=_SC_PARAMS,
    scratch_types=[
        pltpu.VMEM((NCHUNK2, CH), _i32),
        pltpu.VMEM((NCHUNK2, CH), _i32),
        pltpu.VMEM((NCHUNK2, CH), _f32),
        pltpu.VMEM((NPAD,), _f32),
        pltpu.VMEM((CAPB,), _i32),
        pltpu.VMEM((CAPB,), _i32),
        pltpu.VMEM((CAPB,), _f32),
        pltpu.VMEM((L,), _i32),
    ],
)
def _sc_bin(srcp_hbm, dstp_hbm, wp_hbm, dis_hbm,
            csrc_hbm, cdl_hbm, cco_hbm, cnt_hbm,
            sb, db, wb, disb, fsrc, fdl, fco, cntb):
    c = lax.axis_index("c")
    s = lax.axis_index("s")
    g = c * NS + s
    pltpu.sync_copy(srcp_hbm.at[s], sb)
    pltpu.sync_copy(dstp_hbm.at[s], db)
    pltpu.sync_copy(wp_hbm.at[s], wb)
    pltpu.sync_copy(dis_hbm, disb)

    def zf(i, _):
        sl = pl.ds(i * L, L)
        fsrc[sl] = jnp.zeros((L,), _i32)
        fdl[sl] = jnp.zeros((L,), _i32)
        fco[sl] = jnp.zeros((L,), _f32)
        return 0
    lax.fori_loop(0, CAPB // L, zf, 0)

    lo = c * HALF

    def bin_chunk(i, off):
        for k in range(CH // L):
            sl = pl.ds(k * L, L)
            sv = sb[i, sl]
            dv = db[i, sl]
            wv = wb[i, sl]
            dsv = plsc.load_gather(disb, [sv])
            ddv = plsc.load_gather(disb, [dv])
            cv = dsv * wv * ddv
            lv = dv - lo
            own = (lv >= 0) & (lv < HALF)
            plsc.store_compressed(fsrc.at[pl.ds(off, L)], sv, mask=own)
            plsc.store_compressed(fdl.at[pl.ds(off, L)], lv, mask=own)
            plsc.store_compressed(fco.at[pl.ds(off, L)], cv, mask=own)
            cnt16 = plsc.all_reduce_population_count(own)
            off = off + cnt16[0]
        return off
    off = lax.fori_loop(0, NCHUNK2, bin_chunk, jnp.int32(0))

    cntb[...] = jnp.full((L,), off, _i32)
    pltpu.sync_copy(cntb, cnt_hbm.at[pl.ds(g * L, L)])
    pltpu.sync_copy(fsrc.at[pl.ds(0, CAPH)], csrc_hbm.at[pl.ds(g * CAPH, CAPH)])
    pltpu.sync_copy(fdl.at[pl.ds(0, CAPH)], cdl_hbm.at[pl.ds(g * CAPH, CAPH)])
    pltpu.sync_copy(fco.at[pl.ds(0, CAPH)], cco_hbm.at[pl.ds(g * CAPH, CAPH)])


@functools.partial(
    pl.kernel,
    out_type=jax.ShapeDtypeStruct((NC, HALF, HID), _f32),
    mesh=_MESH,
    compiler_params=_SC_PARAMS,
    scratch_types=[
        pltpu.VMEM((CAPH,), _i32),
        pltpu.VMEM((CAPH,), _i32),
        pltpu.VMEM((CAPH,), _f32),
        pltpu.VMEM((NCH3, CH), _i32),
        pltpu.VMEM((CH, HID), _f32),
        pltpu.VMEM((CH, HID), _f32),
        pltpu.VMEM((ZR, HID), _f32),
        pltpu.VMEM((L,), _i32),
        pltpu.VMEM_SHARED((HALF, HID), _f32),
        pltpu.SemaphoreType.DMA,
        pltpu.SemaphoreType.DMA,
    ],
)
def _sc_agg(hm_hbm, csrc_hbm, cdl_hbm, cco_hbm, cnt_hbm, parts_hbm,
            fsrc, fdl, fco, didx2d, rows0, rows1, zbuf, cntb, acc,
            sem0, sem1):
    c = lax.axis_index("c")
    s = lax.axis_index("s")
    g = c * NS + s
    pltpu.sync_copy(csrc_hbm.at[pl.ds(g * CAPH, CAPH)], fsrc)
    pltpu.sync_copy(cdl_hbm.at[pl.ds(g * CAPH, CAPH)], fdl)
    pltpu.sync_copy(cco_hbm.at[pl.ds(g * CAPH, CAPH)], fco)
    pltpu.sync_copy(cnt_hbm.at[pl.ds(g * L, L)], cntb)
    cnt = cntb[...][0]
    npair = (cnt + 2 * CH - 1) // (2 * CH)
    nch = npair * 2

    def cp(i, _):
        for k in range(CH // L):
            didx2d[i, pl.ds(k * L, L)] = fdl[pl.ds(i * CH + k * L, L)]
        return 0
    lax.fori_loop(0, nch, cp, 0)

    def zb(i, _):
        for j in range(HID // L):
            zbuf[i, pl.ds(j * L, L)] = jnp.zeros((L,), _f32)
        return 0
    lax.fori_loop(0, ZR, zb, 0)
    r0 = s * SPT
    for kk in range(SPT // ZR):
        pltpu.sync_copy(zbuf, acc.at[pl.ds(r0 + kk * ZR, ZR)])
    plsc.subcore_barrier()

    def chunk_work(chunk, rowsb):
        def scale(k, _):
            cv = fco[pl.ds(chunk * CH + k * L, L)]
            for r in range(L):
                bv = jnp.full((L,), cv[r], _f32)
                row = k * L + r
                for j in range(HID // L):
                    rowsb[row, pl.ds(j * L, L)] = (
                        rowsb[row, pl.ds(j * L, L)] * bv)
            return 0
        lax.fori_loop(0, CH // L, scale, 0)
        pltpu.sync_copy(rowsb, acc.at[didx2d.at[chunk]], add=True)

    pltpu.async_copy(hm_hbm.at[fsrc.at[pl.ds(0, CH)]], rows0, sem0)

    def pair(i, _):
        c0 = 2 * i
        c2 = jnp.where(c0 + 2 < nch, c0 + 2, 0)
        pltpu.make_async_copy(
            hm_hbm.at[fsrc.at[pl.ds(c0 * CH, CH)]], rows0, sem0).wait()
        pltpu.async_copy(
            hm_hbm.at[fsrc.at[pl.ds((c0 + 1) * CH, CH)]], rows1, sem1)
        chunk_work(c0, rows0)
        pltpu.make_async_copy(
            hm_hbm.at[fsrc.at[pl.ds((c0 + 1) * CH, CH)]], rows1, sem1).wait()
        pltpu.async_copy(
            hm_hbm.at[fsrc.at[pl.ds(c2 * CH, CH)]], rows0, sem0)
        chunk_work(c0 + 1, rows1)
        return 0
    lax.fori_loop(0, npair, pair, 0)
    pltpu.make_async_copy(
        hm_hbm.at[fsrc.at[pl.ds(0, CH)]], rows0, sem0).wait()

    plsc.subcore_barrier()
    pltpu.sync_copy(acc.at[pl.ds(r0, SPT)], parts_hbm.at[c, pl.ds(r0, SPT)])


def kernel(x, edge_index, edge_attr, W1, b1, W2, b2, Wout, bout):
    xf = x.reshape(N, DIN * T)
    w1rep = jnp.repeat(W1, T, axis=0) * (1.0 / T)

    src = edge_index[0].astype(_i32)
    dst = edge_index[1].astype(_i32)
    w = edge_attr[:, -1].astype(_f32)

    loop = jnp.arange(N, dtype=_i32)
    ipad = jnp.zeros((EP - E - N,), dtype=_i32)
    srcf = jnp.concatenate([src, loop, ipad])
    dstf = jnp.concatenate([dst, loop, ipad])
    wf = jnp.concatenate(
        [w, jnp.ones((N,), _f32), jnp.zeros((EP - E - N,), _f32)])

    srcp = srcf.reshape(NS, NCHUNK2, CH)
    dstp = dstf.reshape(NS, NCHUNK2, CH)
    wp = wf.reshape(NS, NCHUNK2, CH)
    dstp_deg = dstf.reshape(NT, NCHUNK, CH)
    wp_deg = wf.reshape(NT, NCHUNK, CH)

    b1r = b1.reshape(1, HID)
    b2r = b2.reshape(1, HID)
    boutr = bout.reshape(1, DOUT)

    hm1 = _tc_matmul(xf, w1rep)
    degp = _sc_deg(dstp_deg, wp_deg)
    dis = _sc_dis(degp)
    csrc, cdl, cco, cnt = _sc_bin(srcp, dstp, wp, dis)

    ws = jnp.stack([W2, Wout])
    bs = jnp.stack([b1r, b2r])
    pbs = jnp.stack([jnp.zeros((1, HID), _f32), boutr])
    flags = jnp.array([[[1.0]], [[0.0]]], _f32)

    def stage(hm, xs):
        wk, bk, pbk, fk = xs
        parts = _sc_agg(hm, csrc, cdl, cco, cnt)
        hm_next = _tc_lin(parts.reshape(NPAD, HID), bk, wk, pbk, fk)
        return hm_next, 0

    out, _ = lax.scan(stage, hm1, (ws, bs, pbs, flags))
    return out[:N]
